# Initial kernel scaffold; baseline (speedup 1.0000x reference)
import jax
import jax.numpy as jnp
from jax.experimental import pallas as pl


def kernel(x):
    raise NotImplementedError("write your pallas kernel here")


# speedup vs baseline: 3.5910x; 3.5910x over previous
import functools

import jax
import jax.numpy as jnp
from jax import lax
from jax.experimental import pallas as pl
from jax.experimental.pallas import tpu as pltpu
from jax.experimental.pallas import tpu_sc as plsc

_ROWS = 128
_W = 16
_RIGHT0 = 16384
_NV = 1024.0
_NWORKERS = 32
_RPW = _ROWS // _NWORKERS


_SLAB = 128


def _mix_body(x_ref, a_v, b_v):
    wid = lax.axis_index("s") * 2 + lax.axis_index("c")
    r0 = wid * _RPW
    pltpu.sync_copy(x_ref.at[pl.ds(r0, _RPW), pl.ds(0, _SLAB)], a_v)
    pltpu.sync_copy(x_ref.at[pl.ds(r0, _RPW), pl.ds(_RIGHT0, _SLAB)], b_v)
    for i in range(_RPW):
        a = a_v[i, pl.ds(0, _W)]
        b = b_v[i, pl.ds(0, _W)]
        a_v[i, pl.ds(0, _W)] = jnp.mod(a + b, _NV) + 1.0
        b_v[i, pl.ds(0, _W)] = jnp.mod(_NV + a - b, _NV) + 1.0
    pltpu.sync_copy(a_v, x_ref.at[pl.ds(r0, _RPW), pl.ds(0, _SLAB)])
    pltpu.sync_copy(b_v, x_ref.at[pl.ds(r0, _RPW), pl.ds(_RIGHT0, _SLAB)])


_mix_fix = functools.partial(
    pl.kernel,
    mesh=plsc.VectorSubcoreMesh(core_axis_name="c", subcore_axis_name="s"),
    scratch_types=[
        pltpu.VMEM((_RPW, _SLAB), jnp.float32),
        pltpu.VMEM((_RPW, _SLAB), jnp.float32),
    ],
)(_mix_body)


def kernel(x):
    ref = jax.new_ref(x[0])
    _mix_fix(ref)
    return (ref[...], x[1], x[2])
